# baseline (speedup 1.0000x reference)
import functools

import jax
import jax.numpy as jnp
from jax import lax
from jax.experimental import pallas as pl
from jax.experimental.pallas import tpu as pltpu
from jax.experimental.pallas import tpu_sc as plsc

PHI_C = 0.01
M_C = 1_000_000
B_C = 131_072
DIM_SH_C = 16

HALF = M_C // 2
NR = 2
QUARTER = HALF // NR
NS = 16
L = 16
UB = B_C // NS
UC = 512
NUC = UB // UC
SCH = 128
ACC_ROWS = QUARTER + NS

ZCH = 640
ZCW = ZCH * 4
NFULL = QUARTER // ZCH
TAIL = QUARTER - NFULL * ZCH
TAILW = TAIL * 4
NJ = (NFULL + NS - 1) // NS
ACC_WORDS = ACC_ROWS * 4

DN = 1250
DNP = 1264
NV = DNP // L
RTD = M_C // (2 * NS)
DCD = RTD // DN


def _rsqrt(s):
    i = plsc.bitcast(s, jnp.int32)
    i = jnp.int32(0x5F3759DF) - (i >> 1)
    y = plsc.bitcast(i, jnp.float32)
    for _ in range(3):
        y = y * (1.5 - 0.5 * s * y * y)
    return y


def _scatter_body(grad, idx, delta4,
                  idx_v, grad_v, vals_v, eff_v, zbuf, dbuf, acc, sem):
    cid = lax.axis_index("c")
    sid = lax.axis_index("s")
    iota = lax.iota(jnp.int32, L)
    c0 = jnp.zeros((L,), jnp.int32)
    c1 = jnp.full((L,), 1, jnp.int32)
    c2 = jnp.full((L,), 2, jnp.int32)
    c3 = jnp.full((L,), 3, jnp.int32)
    zf = jnp.zeros((L,), jnp.float32)

    cp_i = pltpu.async_copy(idx.at[pl.ds(sid * UB, UB)], idx_v, sem)
    cp_g = pltpu.async_copy(grad.at[pl.ds(sid * UB * 3, UB * 3)], grad_v, sem)

    def zb_body(v, _):
        plsc.store_scatter(zbuf, [v * L + iota], zf)
        return 0
    lax.fori_loop(0, ZCW // L, zb_body, 0)

    cp_i.wait()
    cp_g.wait()

    def round_body(r, _):
        qbase = cid * HALF + r * QUARTER

        def z_body(j, _):
            ch = sid + j * NS

            @pl.when(ch < NFULL)
            def _():
                pltpu.sync_copy(zbuf, acc.at[pl.ds(ch * ZCW, ZCW)])
            return 0
        lax.fori_loop(0, NJ, z_body, 0)

        @pl.when(sid == 7)
        def _():
            pltpu.sync_copy(zbuf.at[pl.ds(0, TAILW)],
                            acc.at[pl.ds(NFULL * ZCW, TAILW)])
        plsc.subcore_barrier()

        def u_body(u, _):
            base = u * UC

            def v_body(v, _):
                lrows = v * L + iota
                rows = base + lrows
                ivec = plsc.load_gather(idx_v, [rows])
                p3 = rows * 3
                gx = plsc.load_gather(grad_v, [p3])
                gy = plsc.load_gather(grad_v, [p3 + 1])
                gz = plsc.load_gather(grad_v, [p3 + 2])
                s = gx * gx + gy * gy + gz * gz + 1e-12
                g = s * _rsqrt(s)
                l4 = lrows * 4
                plsc.store_scatter(vals_v, [l4], gx * PHI_C)
                plsc.store_scatter(vals_v, [l4 + 1], gy * PHI_C)
                plsc.store_scatter(vals_v, [l4 + 2], gz * PHI_C)
                plsc.store_scatter(vals_v, [l4 + 3], g)
                tgt = ivec - qbase
                ok = (tgt >= 0) & (tgt < QUARTER)
                w4 = jnp.where(ok, tgt, QUARTER + sid) * 4
                plsc.store_scatter(eff_v, [l4 >> 7, l4 & 127], w4)
                plsc.store_scatter(eff_v, [(l4 + 1) >> 7, (l4 + 1) & 127],
                                   w4 + 1)
                plsc.store_scatter(eff_v, [(l4 + 2) >> 7, (l4 + 2) & 127],
                                   w4 + 2)
                plsc.store_scatter(eff_v, [(l4 + 3) >> 7, (l4 + 3) & 127],
                                   w4 + 3)
                return 0
            lax.fori_loop(0, UC // L, v_body, 0)

            def s_body(q, _):
                pltpu.sync_copy(vals_v.at[pl.ds(q * SCH, SCH)],
                                acc.at[eff_v.at[q]], add=True)
                return 0
            lax.fori_loop(0, UC * 4 // SCH, s_body, 0)
            return 0
        lax.fori_loop(0, NUC, u_body, 0)
        plsc.subcore_barrier()

        qb4 = qbase * 4

        def p_body(j, _):
            ch = sid + j * NS

            @pl.when(ch < NFULL)
            def _():
                pltpu.sync_copy(acc.at[pl.ds(ch * ZCW, ZCW)], dbuf)
                pltpu.sync_copy(dbuf, delta4.at[pl.ds(qb4 + ch * ZCW, ZCW)])
            return 0
        lax.fori_loop(0, NJ, p_body, 0)

        @pl.when(sid == 7)
        def _():
            pltpu.sync_copy(acc.at[pl.ds(NFULL * ZCW, TAILW)],
                            dbuf.at[pl.ds(0, TAILW)])
            pltpu.sync_copy(dbuf.at[pl.ds(0, TAILW)],
                            delta4.at[pl.ds(qb4 + NFULL * ZCW, TAILW)])
        return 0

    lax.fori_loop(0, NR, round_body, 0)


def _dense_body(means, colors, opac, delta4, out,
                mbuf, bbuf, obuf, dbuf, wbuf):
    cid = lax.axis_index("c")
    sid = lax.axis_index("s")
    iota = lax.iota(jnp.int32, L)
    c0 = jnp.zeros((L,), jnp.int32)
    c1 = jnp.full((L,), 1, jnp.int32)
    c2 = jnp.full((L,), 2, jnp.int32)
    c3 = jnp.full((L,), 3, jnp.int32)
    wid = cid * NS + sid

    def d_body(k, _):
        gr0 = wid * RTD + k * DN
        pltpu.sync_copy(means.at[pl.ds(gr0, DN)], mbuf.at[pl.ds(0, DN)])
        pltpu.sync_copy(colors.at[pl.ds(gr0, DN), pl.ds(0, 3)],
                        bbuf.at[pl.ds(0, DN)])
        pltpu.sync_copy(opac.at[pl.ds(gr0, DN)], obuf.at[pl.ds(0, DN)])
        pltpu.sync_copy(delta4.at[pl.ds(gr0 * 4, DN * 4)],
                        dbuf.at[pl.ds(0, DN * 4)])

        def v_body(v, _):
            rows = v * L + iota
            a = plsc.load_gather(obuf, [rows, c0])
            alpha = 1.0 / (1.0 + jnp.exp(-a))
            r4 = rows * 4
            dx = plsc.load_gather(dbuf, [r4])
            dy = plsc.load_gather(dbuf, [r4 + 1])
            dz = plsc.load_gather(dbuf, [r4 + 2])
            ga = plsc.load_gather(dbuf, [r4 + 3])
            mx = plsc.load_gather(mbuf, [rows, c0])
            my = plsc.load_gather(mbuf, [rows, c1])
            mz = plsc.load_gather(mbuf, [rows, c2])
            bx = plsc.load_gather(bbuf, [rows, c0])
            by = plsc.load_gather(bbuf, [rows, c1])
            bz = plsc.load_gather(bbuf, [rows, c2])
            plsc.store_scatter(wbuf, [rows, c0], (mx + dx) * alpha + bx + ga)
            plsc.store_scatter(wbuf, [rows, c1], (my + dy) * alpha + by + ga)
            plsc.store_scatter(wbuf, [rows, c2], (mz + dz) * alpha + bz + ga)
            return 0
        lax.fori_loop(0, NV, v_body, 0)
        pltpu.sync_copy(wbuf.at[pl.ds(0, DN)], out.at[pl.ds(gr0, DN)])
        return 0
    lax.fori_loop(0, DCD, d_body, 0)


def _make_kernels():
    mesh = plsc.VectorSubcoreMesh(core_axis_name="c", subcore_axis_name="s")
    scatter_k = functools.partial(
        pl.kernel,
        out_type=jax.ShapeDtypeStruct((M_C * 4,), jnp.float32),
        mesh=mesh,
        compiler_params=pltpu.CompilerParams(use_tc_tiling_on_sc=True,
                                             needs_layout_passes=False),
        scratch_types=[
            pltpu.VMEM((UB,), jnp.int32),
            pltpu.VMEM((UB * 3,), jnp.float32),
            pltpu.VMEM((UC * 4,), jnp.float32),
            pltpu.VMEM((UC * 4 // SCH, SCH), jnp.int32),
            pltpu.VMEM((ZCW,), jnp.float32),
            pltpu.VMEM((ZCW,), jnp.float32),
            pltpu.VMEM_SHARED((ACC_WORDS,), jnp.float32),
            pltpu.SemaphoreType.DMA,
        ],
    )(_scatter_body)
    dense_k = functools.partial(
        pl.kernel,
        out_type=jax.ShapeDtypeStruct((M_C, 3), jnp.float32),
        mesh=mesh,
        compiler_params=pltpu.CompilerParams(use_tc_tiling_on_sc=False,
                                             needs_layout_passes=False),
        scratch_types=[
            pltpu.VMEM((DNP, 3), jnp.float32),
            pltpu.VMEM((DNP, 3), jnp.float32),
            pltpu.VMEM((DNP, 1), jnp.float32),
            pltpu.VMEM((DNP * 4,), jnp.float32),
            pltpu.VMEM((DNP, 3), jnp.float32),
        ],
    )(_dense_body)
    return scatter_k, dense_k


_scatter_sc, _dense_sc = _make_kernels()


def kernel(means, colors, opacities, grad_vals, idx):
    colors2d = colors.reshape(M_C, DIM_SH_C * 3)
    grad_flat = grad_vals.reshape(B_C * 3)
    delta4 = _scatter_sc(grad_flat, idx)
    return _dense_sc(means, colors2d, opacities, delta4)


# speedup vs baseline: 1.0105x; 1.0105x over previous
import functools

import jax
import jax.numpy as jnp
from jax import lax
from jax.experimental import pallas as pl
from jax.experimental.pallas import tpu as pltpu
from jax.experimental.pallas import tpu_sc as plsc

PHI_C = 0.01
M_C = 1_000_000
B_C = 131_072
DIM_SH_C = 16

HALF = M_C // 2
NR = 2
QUARTER = HALF // NR
NS = 16
L = 16
UB = B_C // NS
UC = 512
NUC = UB // UC
SCH = 128
ACC_ROWS = QUARTER + NS

ZCH = 640
ZCW = ZCH * 4
NFULL = QUARTER // ZCH
TAIL = QUARTER - NFULL * ZCH
TAILW = TAIL * 4
NJ = (NFULL + NS - 1) // NS
ACC_WORDS = ACC_ROWS * 4

DN = 1250
DNP = 1264
NV = DNP // L
RTD = M_C // (2 * NS)
DCD = RTD // DN


def _rsqrt(s):
    i = plsc.bitcast(s, jnp.int32)
    i = jnp.int32(0x5F3759DF) - (i >> 1)
    y = plsc.bitcast(i, jnp.float32)
    for _ in range(3):
        y = y * (1.5 - 0.5 * s * y * y)
    return y


def _scatter_body(grad, idx, delta4,
                  idx_v, grad_v, vals_v, eff_v, zbuf, dbuf, acc, sem):
    cid = lax.axis_index("c")
    sid = lax.axis_index("s")
    iota = lax.iota(jnp.int32, L)
    c0 = jnp.zeros((L,), jnp.int32)
    c1 = jnp.full((L,), 1, jnp.int32)
    c2 = jnp.full((L,), 2, jnp.int32)
    c3 = jnp.full((L,), 3, jnp.int32)
    zf = jnp.zeros((L,), jnp.float32)

    cp_i = pltpu.async_copy(idx.at[pl.ds(sid * UB, UB)], idx_v, sem)
    cp_g = pltpu.async_copy(grad.at[pl.ds(sid * UB * 3, UB * 3)], grad_v, sem)

    def zb_body(v, _):
        plsc.store_scatter(zbuf, [v * L + iota], zf)
        return 0
    lax.fori_loop(0, ZCW // L, zb_body, 0)

    cp_i.wait()
    cp_g.wait()

    def round_body(r, _):
        qbase = cid * HALF + r * QUARTER

        def z_body(j, _):
            ch = sid + j * NS

            @pl.when(ch < NFULL)
            def _():
                pltpu.async_copy(zbuf, acc.at[pl.ds(ch * ZCW, ZCW)], sem)
            return 0
        lax.fori_loop(0, NJ, z_body, 0)

        def z_wait(j, _):
            ch = sid + j * NS

            @pl.when(ch < NFULL)
            def _():
                pltpu.make_async_copy(zbuf,
                                      acc.at[pl.ds(ch * ZCW, ZCW)],
                                      sem).wait()
            return 0
        lax.fori_loop(0, NJ, z_wait, 0)

        @pl.when(sid == 7)
        def _():
            pltpu.sync_copy(zbuf.at[pl.ds(0, TAILW)],
                            acc.at[pl.ds(NFULL * ZCW, TAILW)])
        plsc.subcore_barrier()

        def u_body(u, _):
            base = u * UC

            def v_body(v, _):
                lrows = v * L + iota
                rows = base + lrows
                ivec = plsc.load_gather(idx_v, [rows])
                p3 = rows * 3
                gx = plsc.load_gather(grad_v, [p3])
                gy = plsc.load_gather(grad_v, [p3 + 1])
                gz = plsc.load_gather(grad_v, [p3 + 2])
                s = gx * gx + gy * gy + gz * gz + 1e-12
                g = s * _rsqrt(s)
                l4 = lrows * 4
                plsc.store_scatter(vals_v, [l4], gx * PHI_C)
                plsc.store_scatter(vals_v, [l4 + 1], gy * PHI_C)
                plsc.store_scatter(vals_v, [l4 + 2], gz * PHI_C)
                plsc.store_scatter(vals_v, [l4 + 3], g)
                tgt = ivec - qbase
                ok = (tgt >= 0) & (tgt < QUARTER)
                w4 = jnp.where(ok, tgt, QUARTER + sid) * 4
                plsc.store_scatter(eff_v, [l4 >> 7, l4 & 127], w4)
                plsc.store_scatter(eff_v, [(l4 + 1) >> 7, (l4 + 1) & 127],
                                   w4 + 1)
                plsc.store_scatter(eff_v, [(l4 + 2) >> 7, (l4 + 2) & 127],
                                   w4 + 2)
                plsc.store_scatter(eff_v, [(l4 + 3) >> 7, (l4 + 3) & 127],
                                   w4 + 3)
                return 0
            lax.fori_loop(0, UC // L, v_body, 0)

            def s_start(q, _):
                pltpu.async_copy(vals_v.at[pl.ds(q * SCH, SCH)],
                                 acc.at[eff_v.at[q]], sem, add=True)
                return 0
            lax.fori_loop(0, UC * 4 // SCH, s_start, 0)

            def s_wait(q, _):
                pltpu.make_async_copy(vals_v.at[pl.ds(q * SCH, SCH)],
                                      acc.at[eff_v.at[q]], sem).wait()
                return 0
            lax.fori_loop(0, UC * 4 // SCH, s_wait, 0)
            return 0
        lax.fori_loop(0, NUC, u_body, 0)
        plsc.subcore_barrier()

        qb4 = qbase * 4

        def p_body(j, _):
            ch = sid + j * NS

            @pl.when(ch < NFULL)
            def _():
                pltpu.sync_copy(acc.at[pl.ds(ch * ZCW, ZCW)], dbuf)
                pltpu.sync_copy(dbuf, delta4.at[pl.ds(qb4 + ch * ZCW, ZCW)])
            return 0
        lax.fori_loop(0, NJ, p_body, 0)

        @pl.when(sid == 7)
        def _():
            pltpu.sync_copy(acc.at[pl.ds(NFULL * ZCW, TAILW)],
                            dbuf.at[pl.ds(0, TAILW)])
            pltpu.sync_copy(dbuf.at[pl.ds(0, TAILW)],
                            delta4.at[pl.ds(qb4 + NFULL * ZCW, TAILW)])
        return 0

    lax.fori_loop(0, NR, round_body, 0)


def _dense_body(means, colors, opac, delta4, out,
                mbuf, bbuf, obuf, dbuf, wbuf, sem, osem):
    cid = lax.axis_index("c")
    sid = lax.axis_index("s")
    iota = lax.iota(jnp.int32, L)
    c0 = jnp.zeros((L,), jnp.int32)
    c1 = jnp.full((L,), 1, jnp.int32)
    c2 = jnp.full((L,), 2, jnp.int32)
    c3 = jnp.full((L,), 3, jnp.int32)
    wid = cid * NS + sid

    def d_body(k, _):
        gr0 = wid * RTD + k * DN
        cpm = pltpu.async_copy(means.at[pl.ds(gr0, DN)],
                               mbuf.at[pl.ds(0, DN)], sem)
        cpb = pltpu.async_copy(colors.at[pl.ds(gr0, DN), pl.ds(0, 3)],
                               bbuf.at[pl.ds(0, DN)], sem)
        cpo = pltpu.async_copy(opac.at[pl.ds(gr0, DN)],
                               obuf.at[pl.ds(0, DN)], sem)
        cpd = pltpu.async_copy(delta4.at[pl.ds(gr0 * 4, DN * 4)],
                               dbuf.at[pl.ds(0, DN * 4)], sem)
        cpm.wait()
        cpb.wait()
        cpo.wait()
        cpd.wait()

        @pl.when(k > 0)
        def _():
            gp = wid * RTD + (k - 1) * DN
            pltpu.make_async_copy(wbuf.at[pl.ds(0, DN)],
                                  out.at[pl.ds(gp, DN)], osem).wait()

        def v_body(v, _):
            rows = v * L + iota
            a = plsc.load_gather(obuf, [rows, c0])
            alpha = 1.0 / (1.0 + jnp.exp(-a))
            r4 = rows * 4
            dx = plsc.load_gather(dbuf, [r4])
            dy = plsc.load_gather(dbuf, [r4 + 1])
            dz = plsc.load_gather(dbuf, [r4 + 2])
            ga = plsc.load_gather(dbuf, [r4 + 3])
            mx = plsc.load_gather(mbuf, [rows, c0])
            my = plsc.load_gather(mbuf, [rows, c1])
            mz = plsc.load_gather(mbuf, [rows, c2])
            bx = plsc.load_gather(bbuf, [rows, c0])
            by = plsc.load_gather(bbuf, [rows, c1])
            bz = plsc.load_gather(bbuf, [rows, c2])
            plsc.store_scatter(wbuf, [rows, c0], (mx + dx) * alpha + bx + ga)
            plsc.store_scatter(wbuf, [rows, c1], (my + dy) * alpha + by + ga)
            plsc.store_scatter(wbuf, [rows, c2], (mz + dz) * alpha + bz + ga)
            return 0
        lax.fori_loop(0, NV, v_body, 0)
        pltpu.async_copy(wbuf.at[pl.ds(0, DN)], out.at[pl.ds(gr0, DN)], osem)
        return 0
    lax.fori_loop(0, DCD, d_body, 0)
    gl = wid * RTD + (DCD - 1) * DN
    pltpu.make_async_copy(wbuf.at[pl.ds(0, DN)],
                          out.at[pl.ds(gl, DN)], osem).wait()


def _make_kernels():
    mesh = plsc.VectorSubcoreMesh(core_axis_name="c", subcore_axis_name="s")
    scatter_k = functools.partial(
        pl.kernel,
        out_type=jax.ShapeDtypeStruct((M_C * 4,), jnp.float32),
        mesh=mesh,
        compiler_params=pltpu.CompilerParams(use_tc_tiling_on_sc=True,
                                             needs_layout_passes=False),
        scratch_types=[
            pltpu.VMEM((UB,), jnp.int32),
            pltpu.VMEM((UB * 3,), jnp.float32),
            pltpu.VMEM((UC * 4,), jnp.float32),
            pltpu.VMEM((UC * 4 // SCH, SCH), jnp.int32),
            pltpu.VMEM((ZCW,), jnp.float32),
            pltpu.VMEM((ZCW,), jnp.float32),
            pltpu.VMEM_SHARED((ACC_WORDS,), jnp.float32),
            pltpu.SemaphoreType.DMA,
        ],
    )(_scatter_body)
    dense_k = functools.partial(
        pl.kernel,
        out_type=jax.ShapeDtypeStruct((M_C, 3), jnp.float32),
        mesh=mesh,
        compiler_params=pltpu.CompilerParams(use_tc_tiling_on_sc=False,
                                             needs_layout_passes=False),
        scratch_types=[
            pltpu.VMEM((DNP, 3), jnp.float32),
            pltpu.VMEM((DNP, 3), jnp.float32),
            pltpu.VMEM((DNP, 1), jnp.float32),
            pltpu.VMEM((DNP * 4,), jnp.float32),
            pltpu.VMEM((DNP, 3), jnp.float32),
            pltpu.SemaphoreType.DMA,
            pltpu.SemaphoreType.DMA,
        ],
    )(_dense_body)
    return scatter_k, dense_k


_scatter_sc, _dense_sc = _make_kernels()


def kernel(means, colors, opacities, grad_vals, idx):
    colors2d = colors.reshape(M_C, DIM_SH_C * 3)
    grad_flat = grad_vals.reshape(B_C * 3)
    delta4 = _scatter_sc(grad_flat, idx)
    return _dense_sc(means, colors2d, opacities, delta4)
